# baseline (speedup 1.0000x reference)
import functools

import jax
import jax.numpy as jnp
from jax import lax
from jax.experimental import pallas as pl
from jax.experimental.pallas import tpu as pltpu
from jax.experimental.pallas import tpu_sc as plsc

B = 16384
EMB = 32
NC = 2
NS = 16
NW = NC * NS
BW = B // NW
CHUNK = 128
NCH = BW // CHUNK
L = 16
NG = BW // L


def _mf_body(uid_hbm, iid_hbm, ue_hbm, ie_hbm, ub_hbm, ib_hbm, gb_hbm,
             out_hbm,
             uid_v, iid_v, ue_v, ie_v, ub_v, ib_v, gb_v, out_v, sem):
    wid = lax.axis_index("s") * NC + lax.axis_index("c")

    pltpu.sync_copy(uid_hbm.at[wid], uid_v)
    pltpu.sync_copy(iid_hbm.at[wid], iid_v)
    pltpu.sync_copy(gb_hbm, gb_v)

    copies = []
    for c in range(NCH):
        rows = pl.ds(c * CHUNK, CHUNK)
        copies.append(pltpu.async_copy(ue_hbm.at[uid_v.at[c]], ue_v.at[rows], sem))
        copies.append(pltpu.async_copy(ie_hbm.at[iid_v.at[c]], ie_v.at[rows], sem))
        copies.append(pltpu.async_copy(ub_hbm.at[uid_v.at[c]], ub_v.at[rows], sem))
        copies.append(pltpu.async_copy(ib_hbm.at[iid_v.at[c]], ib_v.at[rows], sem))
    for cp in copies:
        cp.wait()

    gb = gb_v[...]
    iota = lax.iota(jnp.int32, L)

    def group(g, _):
        o = pl.ds(g * L, L)
        rows = g * L + iota
        acc = gb + ub_v[o] + ib_v[o]
        for j in range(EMB):
            cols = jnp.full((L,), j, jnp.int32)
            u = plsc.load_gather(ue_v, [rows, cols])
            i = plsc.load_gather(ie_v, [rows, cols])
            acc = acc + u * i
        out_v[o] = acc
        return 0

    lax.fori_loop(0, NG, group, 0)
    pltpu.sync_copy(out_v, out_hbm.at[pl.ds(wid * BW, BW)])


@functools.partial(jax.jit, static_argnames=())
def kernel(user_id, item_id, user_embedding, item_embedding, user_bias,
           item_bias, global_bias):
    uid3 = user_id.astype(jnp.int32).reshape(NW, NCH, CHUNK)
    iid3 = item_id.astype(jnp.int32).reshape(NW, NCH, CHUNK)
    ub_flat = user_bias.reshape(-1)
    ib_flat = item_bias.reshape(-1)
    gb16 = jnp.broadcast_to(global_bias, (L,)).astype(jnp.float32)

    run = pl.kernel(
        _mf_body,
        out_type=jax.ShapeDtypeStruct((B,), jnp.float32),
        mesh=plsc.VectorSubcoreMesh(
            core_axis_name="c", subcore_axis_name="s",
            num_cores=NC, num_subcores=NS),
        scratch_types=[
            pltpu.VMEM((NCH, CHUNK), jnp.int32),
            pltpu.VMEM((NCH, CHUNK), jnp.int32),
            pltpu.VMEM((BW, EMB), jnp.float32),
            pltpu.VMEM((BW, EMB), jnp.float32),
            pltpu.VMEM((BW,), jnp.float32),
            pltpu.VMEM((BW,), jnp.float32),
            pltpu.VMEM((L,), jnp.float32),
            pltpu.VMEM((BW,), jnp.float32),
            pltpu.SemaphoreType.DMA,
        ],
        compiler_params=pltpu.CompilerParams(
            needs_layout_passes=False, use_tc_tiling_on_sc=False),
    )
    return run(uid3, iid3, user_embedding, item_embedding, ub_flat, ib_flat,
               gb16)
